# baseline (speedup 1.0000x reference)
import functools

import jax
import jax.numpy as jnp
from jax import lax
from jax.experimental import pallas as pl
from jax.experimental.pallas import tpu as pltpu
from jax.experimental.pallas import tpu_sc as plsc

N = 10000
E = 320000
F_IN = 128
HID = 256
N_EXP = 4
N_GRAPH = 64
N_CLS = 10

NC = 2
NS = 16
CH = 128
MC = 8
NTRASH = 64
RS = 632

NB = 10
BLK = N // NB


def _make_sc_agg(nm):
  mesh = plsc.VectorSubcoreMesh(
      core_axis_name="c", subcore_axis_name="s", num_cores=NC, num_subcores=NS)
  tch = NS * nm * MC

  def body(tbl, srcf, dstf, zeros, out, sbuf, dbuf, rows, acc, gsem, isem,
           ssem):
    c = lax.axis_index("c")
    s = lax.axis_index("s")
    tail0 = (NS - 1) * RS
    tail_n = N - tail0

    def copy_rows(mk_src, mk_dst):
      r0 = s * RS

      @pl.when(s < NS - 1)
      def _():
        pltpu.sync_copy(mk_src(r0, RS), mk_dst(r0, RS))

      @pl.when(s == NS - 1)
      def _():
        pltpu.sync_copy(mk_src(tail0, tail_n), mk_dst(tail0, tail_n))

    copy_rows(lambda r, n: zeros.at[pl.ds(r, n)],
              lambda r, n: acc.at[pl.ds(r, n)])
    plsc.subcore_barrier()

    def idx_row(m):
      return c * tch + s * (nm * MC) + m * MC

    def load_idx(m, b):
      r = idx_row(m)
      pltpu.async_copy(srcf.at[pl.ds(r, MC)], sbuf.at[pl.ds(b * MC, MC)],
                       isem)
      pltpu.async_copy(dstf.at[pl.ds(r, MC)], dbuf.at[pl.ds(b * MC, MC)],
                       isem)

    def wait_idx(b):
      pltpu.make_async_copy(srcf.at[pl.ds(0, MC)],
                            sbuf.at[pl.ds(b * MC, MC)], isem).wait()
      pltpu.make_async_copy(dstf.at[pl.ds(0, MC)],
                            dbuf.at[pl.ds(b * MC, MC)], isem).wait()

    load_idx(0, 0)

    def drain_scatter():
      pltpu.make_async_copy(rows.at[0], acc.at[dbuf.at[0]], ssem).wait()

    def macro(m, carry):
      b = m % 2
      wait_idx(b)

      @pl.when(m > 0)
      def _():
        drain_scatter()

      @pl.when(m < nm - 1)
      def _():
        load_idx(m + 1, 1 - b)

      gd = [None] * MC
      sd = [None] * MC
      gd[0] = pltpu.async_copy(tbl.at[sbuf.at[b * MC]], rows.at[0], gsem)
      for k in range(MC):
        gd[k].wait()
        if k >= 1:
          sd[k - 1].wait()
        if k + 1 < MC:
          gd[k + 1] = pltpu.async_copy(tbl.at[sbuf.at[b * MC + k + 1]],
                                       rows.at[(k + 1) % 2], gsem)
        sd[k] = pltpu.async_copy(rows.at[k % 2], acc.at[dbuf.at[b * MC + k]],
                                 ssem, add=True)
      return carry

    lax.fori_loop(0, nm, macro, 0)
    drain_scatter()
    plsc.subcore_barrier()
    copy_rows(lambda r, n: acc.at[pl.ds(r, n)],
              lambda r, n: out.at[c, pl.ds(r, n)])

  return pl.kernel(
      body,
      out_type=jax.ShapeDtypeStruct((NC, N, 128), jnp.float32),
      mesh=mesh,
      scratch_types=[
          pltpu.VMEM((2 * MC, CH), jnp.int32),
          pltpu.VMEM((2 * MC, CH), jnp.int32),
          pltpu.VMEM((2, CH, 128), jnp.float32),
          pltpu.VMEM_SHARED((N + NTRASH, 128), jnp.float32),
          pltpu.SemaphoreType.DMA,
          pltpu.SemaphoreType.DMA,
          pltpu.SemaphoreType.DMA,
      ],
  )


NM_EDGE = 10
NM_FEAT = 20
_sc_agg_edge = _make_sc_agg(NM_EDGE)
_sc_agg_feat = _make_sc_agg(NM_FEAT)


def _pad_idx(parts_src, parts_dst, per_core):
  pad = per_core - parts_src[0][0].shape[0]
  ar = jnp.arange(pad, dtype=jnp.int32)
  psrc = (ar * 131) % N
  pdst = N + (ar % NTRASH)
  srcf = jnp.concatenate(
      [jnp.concatenate([p, psrc + off]) for p, off in parts_src])
  dstf = jnp.concatenate([jnp.concatenate([p, pdst]) for p in parts_dst])
  return (srcf.reshape(-1, CH), dstf.reshape(-1, CH))


def _mlp(z, w1, b1, w2, b2):
  a = jnp.maximum(jnp.dot(z, w1, preferred_element_type=jnp.float32) + b1, 0.0)
  return jnp.maximum(jnp.dot(a, w2, preferred_element_type=jnp.float32) + b2,
                     0.0)


def _l0_body(eps_r, x_r, agg_r, w1_r, b1_r, w2_r, b2_r, out_r):
  z = x_r[...] * (1.0 + eps_r[0, 0]) + agg_r[0] + agg_r[1]
  o = _mlp(z, w1_r[...], b1_r[...], w2_r[...], b2_r[...])
  out_r[0] = o[:, :128]
  out_r[1] = o[:, 128:]


def _mid_body(eps_r, h_r, agg_r, w1_r, b1_r, w2_r, b2_r, out_r):
  h = jnp.concatenate([h_r[0], h_r[1]], axis=1)
  ag = jnp.concatenate([agg_r[0], agg_r[1]], axis=1)
  o = _mlp(h * (1.0 + eps_r[0, 0]) + ag, w1_r[...], b1_r[...], w2_r[...],
           b2_r[...])
  out_r[0] = o[:, :128]
  out_r[1] = o[:, 128:]


def _last_body(eps_r, h_r, agg_r, b_r, w1_r, b1_r, w2_r, b2_r,
               pooled_r, cnt_r):
  i = pl.program_id(0)
  h = jnp.concatenate([h_r[0], h_r[1]], axis=1)
  ag = jnp.concatenate([agg_r[0], agg_r[1]], axis=1)
  o = _mlp(h * (1.0 + eps_r[0, 0]) + ag, w1_r[...], b1_r[...], w2_r[...],
           b2_r[...])
  gids = lax.broadcasted_iota(jnp.int32, (N_GRAPH, 1), 0)
  oh = (gids == b_r[0]).astype(jnp.float32)
  ps = jnp.dot(oh, o, preferred_element_type=jnp.float32)
  cs = jnp.broadcast_to(jnp.sum(oh, axis=1, keepdims=True), (N_GRAPH, 128))

  @pl.when(i == 0)
  def _():
    pooled_r[...] = ps
    cnt_r[...] = cs

  @pl.when(i > 0)
  def _():
    pooled_r[...] += ps
    cnt_r[...] += cs


_smem11 = pl.BlockSpec(memory_space=pltpu.SMEM)
_half_spec = pl.BlockSpec((2, BLK, 128), lambda i: (0, i, 0))


def _tc_l0(eps_e, x, agg, w1, b1, w2, b2):
  return pl.pallas_call(
      _l0_body,
      grid=(NB,),
      in_specs=[
          _smem11,
          pl.BlockSpec((BLK, 128), lambda i: (i, 0)),
          _half_spec,
          pl.BlockSpec((128, HID), lambda i: (0, 0)),
          pl.BlockSpec((1, HID), lambda i: (0, 0)),
          pl.BlockSpec((HID, HID), lambda i: (0, 0)),
          pl.BlockSpec((1, HID), lambda i: (0, 0)),
      ],
      out_specs=_half_spec,
      out_shape=jax.ShapeDtypeStruct((2, N, 128), jnp.float32),
  )(eps_e, x, agg, w1, b1, w2, b2)


def _tc_mid(eps_e, h, agg, w1, b1, w2, b2):
  return pl.pallas_call(
      _mid_body,
      grid=(NB,),
      in_specs=[
          _smem11,
          _half_spec,
          _half_spec,
          pl.BlockSpec((HID, HID), lambda i: (0, 0)),
          pl.BlockSpec((1, HID), lambda i: (0, 0)),
          pl.BlockSpec((HID, HID), lambda i: (0, 0)),
          pl.BlockSpec((1, HID), lambda i: (0, 0)),
      ],
      out_specs=_half_spec,
      out_shape=jax.ShapeDtypeStruct((2, N, 128), jnp.float32),
  )(eps_e, h, agg, w1, b1, w2, b2)


def _tc_last(eps_e, h, agg, batch3, w1, b1, w2, b2):
  return pl.pallas_call(
      _last_body,
      grid=(NB,),
      in_specs=[
          _smem11,
          _half_spec,
          _half_spec,
          pl.BlockSpec((1, 1, BLK), lambda i: (i, 0, 0)),
          pl.BlockSpec((HID, HID), lambda i: (0, 0)),
          pl.BlockSpec((1, HID), lambda i: (0, 0)),
          pl.BlockSpec((HID, HID), lambda i: (0, 0)),
          pl.BlockSpec((1, HID), lambda i: (0, 0)),
      ],
      out_specs=[
          pl.BlockSpec((N_GRAPH, HID), lambda i: (0, 0)),
          pl.BlockSpec((N_GRAPH, 128), lambda i: (0, 0)),
      ],
      out_shape=[
          jax.ShapeDtypeStruct((N_GRAPH, HID), jnp.float32),
          jax.ShapeDtypeStruct((N_GRAPH, 128), jnp.float32),
      ],
  )(eps_e, h, agg, batch3, w1, b1, w2, b2)


def _head_body(cnt_r, wo_r, bo_r, p0, p1, p2, p3, out_r):
  inv = 1.0 / jnp.maximum(cnt_r[...][:, 0:1], 1.0)
  bo = bo_r[...]
  acc = jnp.zeros((N_GRAPH, N_CLS), jnp.float32)
  for e, p in enumerate((p0, p1, p2, p3)):
    acc = acc + jnp.dot(p[...] * inv, wo_r[e],
                        preferred_element_type=jnp.float32) + bo[e:e + 1, :]
  out_r[...] = acc * 0.25


def _head(cnt, wout, bout, pooled):
  return pl.pallas_call(
      _head_body,
      out_shape=jax.ShapeDtypeStruct((N_GRAPH, N_CLS), jnp.float32),
  )(cnt, wout, bout, *pooled)


def kernel(x, edge_index, batch, W1_0, b1_0, W2_0, b2_0, W1_1, b1_1, W2_1,
           b2_1, W1_2, b1_2, W2_2, b2_2, eps, Wout, bout):
  src = edge_index[0]
  dst = edge_index[1]
  half = E // NC
  src_e, dst_e = _pad_idx(
      [(src[:half], 0), (src[half:], 0)],
      [dst[:half], dst[half:]], NS * NM_EDGE * MC * CH)
  src_f, dst_f = _pad_idx(
      [(src, 0), (src + N, N)],
      [dst, dst], NS * NM_FEAT * MC * CH)
  zeros = jnp.zeros((N, 128), jnp.float32)
  batch3 = batch.reshape(NB, 1, BLK)

  agg0 = _sc_agg_edge(x, src_e, dst_e, zeros)

  h1, h2, agg1, agg2, pooled = [], [], [], [], []
  cnt = None
  for e in range(N_EXP):
    h1.append(_tc_l0(eps[e, 0].reshape(1, 1), x, agg0, W1_0[e],
                     b1_0[e].reshape(1, HID), W2_0[e], b2_0[e].reshape(1, HID)))
  for e in range(N_EXP):
    agg1.append(_sc_agg_feat(h1[e].reshape(2 * N, 128), src_f, dst_f, zeros))
  for e in range(N_EXP):
    h2.append(_tc_mid(eps[e, 1].reshape(1, 1), h1[e], agg1[e], W1_1[e],
                      b1_1[e].reshape(1, HID), W2_1[e],
                      b2_1[e].reshape(1, HID)))
  for e in range(N_EXP):
    agg2.append(_sc_agg_feat(h2[e].reshape(2 * N, 128), src_f, dst_f, zeros))
  for e in range(N_EXP):
    p, c = _tc_last(eps[e, 2].reshape(1, 1), h2[e], agg2[e], batch3, W1_2[e],
                    b1_2[e].reshape(1, HID), W2_2[e], b2_2[e].reshape(1, HID))
    pooled.append(p)
    if cnt is None:
      cnt = c
  return _head(cnt, Wout, bout, pooled)


# speedup vs baseline: 1.4351x; 1.4351x over previous
import functools

import jax
import jax.numpy as jnp
from jax import lax
from jax.experimental import pallas as pl
from jax.experimental.pallas import tpu as pltpu
from jax.experimental.pallas import tpu_sc as plsc

N = 10000
E = 320000
F_IN = 128
HID = 256
N_EXP = 4
N_GRAPH = 64
N_CLS = 10

NC = 2
NS = 16
CH = 128
MC = 8
NTRASH = 64
RS = 632

NB = 10
BLK = N // NB


def _make_sc_agg(npc):
  mesh = plsc.VectorSubcoreMesh(
      core_axis_name="c", subcore_axis_name="s", num_cores=NC, num_subcores=NS)

  def body(tbl, srcf, dstf, zeros, out, sbuf, dbuf, rows, acc, gsem, isem,
           ssem):
    c = lax.axis_index("c")
    s = lax.axis_index("s")
    tail0 = (NS - 1) * RS
    tail_n = N - tail0

    def copy_rows(mk_src, mk_dst):
      r0 = s * RS

      @pl.when(s < NS - 1)
      def _():
        pltpu.sync_copy(mk_src(r0, RS), mk_dst(r0, RS))

      @pl.when(s == NS - 1)
      def _():
        pltpu.sync_copy(mk_src(tail0, tail_n), mk_dst(tail0, tail_n))

    copy_rows(lambda r, n: zeros.at[pl.ds(r, n)],
              lambda r, n: acc.at[pl.ds(r, n)])
    plsc.subcore_barrier()

    base = (c * NS + s) * npc * CH

    def load_idx(j, jm4):
      off = base + j * CH
      pltpu.async_copy(srcf.at[pl.ds(off, CH)], sbuf.at[jm4], isem)
      pltpu.async_copy(dstf.at[pl.ds(off, CH)], dbuf.at[jm4], isem)

    def wait_idx_pair():
      pltpu.make_async_copy(srcf.at[pl.ds(0, CH)], sbuf.at[0], isem).wait()
      pltpu.make_async_copy(dstf.at[pl.ds(0, CH)], dbuf.at[0], isem).wait()

    def gather(jm4, jm3):
      pltpu.async_copy(tbl.at[sbuf.at[jm4]], rows.at[jm3], gsem)

    def wait_gather():
      pltpu.make_async_copy(tbl.at[sbuf.at[0]], rows.at[0], gsem).wait()

    def scatter(jm4, jm3):
      pltpu.async_copy(rows.at[jm3], acc.at[dbuf.at[jm4]], ssem, add=True)

    def wait_scatter():
      pltpu.make_async_copy(rows.at[0], acc.at[dbuf.at[0]], ssem).wait()

    load_idx(0, 0)
    wait_idx_pair()
    gather(0, 0)
    load_idx(1, 1)
    wait_idx_pair()
    gather(1, 1)
    load_idx(2, 2)

    def it(j, carry):
      wait_gather()

      @pl.when(j >= 1)
      def _():
        wait_scatter()

      @pl.when(j + 2 < npc)
      def _():
        wait_idx_pair()
        gather((j + 2) % 4, (j + 2) % 3)

      @pl.when(j + 3 < npc)
      def _():
        load_idx(j + 3, (j + 3) % 4)

      scatter(j % 4, j % 3)
      return carry

    lax.fori_loop(0, npc, it, 0)
    wait_scatter()
    plsc.subcore_barrier()
    copy_rows(lambda r, n: acc.at[pl.ds(r, n)],
              lambda r, n: out.at[c, pl.ds(r, n)])

  return pl.kernel(
      body,
      out_type=jax.ShapeDtypeStruct((NC, N, 128), jnp.float32),
      mesh=mesh,
      scratch_types=[
          pltpu.VMEM((4, CH), jnp.int32),
          pltpu.VMEM((4, CH), jnp.int32),
          pltpu.VMEM((3, CH, 128), jnp.float32),
          pltpu.VMEM_SHARED((N + NTRASH, 128), jnp.float32),
          pltpu.SemaphoreType.DMA,
          pltpu.SemaphoreType.DMA,
          pltpu.SemaphoreType.DMA,
      ],
  )


NPC_EDGE = 80
NPC_FEAT = 160
_sc_agg_edge = _make_sc_agg(NPC_EDGE)
_sc_agg_feat = _make_sc_agg(NPC_FEAT)


def _pad_idx(parts_src, parts_dst, per_core):
  pad = per_core - parts_src[0][0].shape[0]
  ar = jnp.arange(pad, dtype=jnp.int32)
  psrc = (ar * 131) % N
  pdst = N + (ar % NTRASH)
  srcf = jnp.concatenate(
      [jnp.concatenate([p, psrc + off]) for p, off in parts_src])
  dstf = jnp.concatenate([jnp.concatenate([p, pdst]) for p in parts_dst])
  return (srcf, dstf)


def _mlp(z, w1, b1, w2, b2):
  a = jnp.maximum(jnp.dot(z, w1, preferred_element_type=jnp.float32) + b1, 0.0)
  return jnp.maximum(jnp.dot(a, w2, preferred_element_type=jnp.float32) + b2,
                     0.0)


def _l0_body(eps_r, x_r, agg_r, w1_r, b1_r, w2_r, b2_r, out_r):
  z = x_r[...] * (1.0 + eps_r[0, 0]) + agg_r[0] + agg_r[1]
  o = _mlp(z, w1_r[...], b1_r[...], w2_r[...], b2_r[...])
  out_r[0] = o[:, :128]
  out_r[1] = o[:, 128:]


def _mid_body(eps_r, h_r, agg_r, w1_r, b1_r, w2_r, b2_r, out_r):
  h = jnp.concatenate([h_r[0], h_r[1]], axis=1)
  ag = jnp.concatenate([agg_r[0], agg_r[1]], axis=1)
  o = _mlp(h * (1.0 + eps_r[0, 0]) + ag, w1_r[...], b1_r[...], w2_r[...],
           b2_r[...])
  out_r[0] = o[:, :128]
  out_r[1] = o[:, 128:]


def _last_body(eps_r, h_r, agg_r, b_r, w1_r, b1_r, w2_r, b2_r,
               pooled_r, cnt_r):
  i = pl.program_id(0)
  h = jnp.concatenate([h_r[0], h_r[1]], axis=1)
  ag = jnp.concatenate([agg_r[0], agg_r[1]], axis=1)
  o = _mlp(h * (1.0 + eps_r[0, 0]) + ag, w1_r[...], b1_r[...], w2_r[...],
           b2_r[...])
  gids = lax.broadcasted_iota(jnp.int32, (N_GRAPH, 1), 0)
  oh = (gids == b_r[0]).astype(jnp.float32)
  ps = jnp.dot(oh, o, preferred_element_type=jnp.float32)
  cs = jnp.broadcast_to(jnp.sum(oh, axis=1, keepdims=True), (N_GRAPH, 128))

  @pl.when(i == 0)
  def _():
    pooled_r[...] = ps
    cnt_r[...] = cs

  @pl.when(i > 0)
  def _():
    pooled_r[...] += ps
    cnt_r[...] += cs


_smem11 = pl.BlockSpec(memory_space=pltpu.SMEM)
_half_spec = pl.BlockSpec((2, BLK, 128), lambda i: (0, i, 0))


def _tc_l0(eps_e, x, agg, w1, b1, w2, b2):
  return pl.pallas_call(
      _l0_body,
      grid=(NB,),
      in_specs=[
          _smem11,
          pl.BlockSpec((BLK, 128), lambda i: (i, 0)),
          _half_spec,
          pl.BlockSpec((128, HID), lambda i: (0, 0)),
          pl.BlockSpec((1, HID), lambda i: (0, 0)),
          pl.BlockSpec((HID, HID), lambda i: (0, 0)),
          pl.BlockSpec((1, HID), lambda i: (0, 0)),
      ],
      out_specs=_half_spec,
      out_shape=jax.ShapeDtypeStruct((2, N, 128), jnp.float32),
  )(eps_e, x, agg, w1, b1, w2, b2)


def _tc_mid(eps_e, h, agg, w1, b1, w2, b2):
  return pl.pallas_call(
      _mid_body,
      grid=(NB,),
      in_specs=[
          _smem11,
          _half_spec,
          _half_spec,
          pl.BlockSpec((HID, HID), lambda i: (0, 0)),
          pl.BlockSpec((1, HID), lambda i: (0, 0)),
          pl.BlockSpec((HID, HID), lambda i: (0, 0)),
          pl.BlockSpec((1, HID), lambda i: (0, 0)),
      ],
      out_specs=_half_spec,
      out_shape=jax.ShapeDtypeStruct((2, N, 128), jnp.float32),
  )(eps_e, h, agg, w1, b1, w2, b2)


def _tc_last(eps_e, h, agg, batch3, w1, b1, w2, b2):
  return pl.pallas_call(
      _last_body,
      grid=(NB,),
      in_specs=[
          _smem11,
          _half_spec,
          _half_spec,
          pl.BlockSpec((1, 1, BLK), lambda i: (i, 0, 0)),
          pl.BlockSpec((HID, HID), lambda i: (0, 0)),
          pl.BlockSpec((1, HID), lambda i: (0, 0)),
          pl.BlockSpec((HID, HID), lambda i: (0, 0)),
          pl.BlockSpec((1, HID), lambda i: (0, 0)),
      ],
      out_specs=[
          pl.BlockSpec((N_GRAPH, HID), lambda i: (0, 0)),
          pl.BlockSpec((N_GRAPH, 128), lambda i: (0, 0)),
      ],
      out_shape=[
          jax.ShapeDtypeStruct((N_GRAPH, HID), jnp.float32),
          jax.ShapeDtypeStruct((N_GRAPH, 128), jnp.float32),
      ],
  )(eps_e, h, agg, batch3, w1, b1, w2, b2)


def _head_body(cnt_r, wo_r, bo_r, p0, p1, p2, p3, out_r):
  inv = 1.0 / jnp.maximum(cnt_r[...][:, 0:1], 1.0)
  bo = bo_r[...]
  acc = jnp.zeros((N_GRAPH, N_CLS), jnp.float32)
  for e, p in enumerate((p0, p1, p2, p3)):
    acc = acc + jnp.dot(p[...] * inv, wo_r[e],
                        preferred_element_type=jnp.float32) + bo[e:e + 1, :]
  out_r[...] = acc * 0.25


def _head(cnt, wout, bout, pooled):
  return pl.pallas_call(
      _head_body,
      out_shape=jax.ShapeDtypeStruct((N_GRAPH, N_CLS), jnp.float32),
  )(cnt, wout, bout, *pooled)


def kernel(x, edge_index, batch, W1_0, b1_0, W2_0, b2_0, W1_1, b1_1, W2_1,
           b2_1, W1_2, b1_2, W2_2, b2_2, eps, Wout, bout):
  src = edge_index[0]
  dst = edge_index[1]
  half = E // NC
  src_e, dst_e = _pad_idx(
      [(src[:half], 0), (src[half:], 0)],
      [dst[:half], dst[half:]], NS * NPC_EDGE * CH)
  src_f, dst_f = _pad_idx(
      [(src, 0), (src + N, N)],
      [dst, dst], NS * NPC_FEAT * CH)
  zeros = jnp.zeros((N, 128), jnp.float32)
  batch3 = batch.reshape(NB, 1, BLK)

  agg0 = _sc_agg_edge(x, src_e, dst_e, zeros)

  h1, h2, agg1, agg2, pooled = [], [], [], [], []
  cnt = None
  for e in range(N_EXP):
    h1.append(_tc_l0(eps[e, 0].reshape(1, 1), x, agg0, W1_0[e],
                     b1_0[e].reshape(1, HID), W2_0[e], b2_0[e].reshape(1, HID)))
  for e in range(N_EXP):
    agg1.append(_sc_agg_feat(h1[e].reshape(2 * N, 128), src_f, dst_f, zeros))
  for e in range(N_EXP):
    h2.append(_tc_mid(eps[e, 1].reshape(1, 1), h1[e], agg1[e], W1_1[e],
                      b1_1[e].reshape(1, HID), W2_1[e],
                      b2_1[e].reshape(1, HID)))
  for e in range(N_EXP):
    agg2.append(_sc_agg_feat(h2[e].reshape(2 * N, 128), src_f, dst_f, zeros))
  for e in range(N_EXP):
    p, c = _tc_last(eps[e, 2].reshape(1, 1), h2[e], agg2[e], batch3, W1_2[e],
                    b1_2[e].reshape(1, HID), W2_2[e], b2_2[e].reshape(1, HID))
    pooled.append(p)
    if cnt is None:
      cnt = c
  return _head(cnt, Wout, bout, pooled)


# speedup vs baseline: 1.4619x; 1.0187x over previous
import functools

import jax
import jax.numpy as jnp
from jax import lax
from jax.experimental import pallas as pl
from jax.experimental.pallas import tpu as pltpu
from jax.experimental.pallas import tpu_sc as plsc

N = 10000
E = 320000
F_IN = 128
HID = 256
N_EXP = 4
N_GRAPH = 64
N_CLS = 10

NC = 2
NS = 16
CH = 88
NTRASH = 64
RS = 632

NB = 10
BLK = N // NB


def _make_sc_agg(npc):
  mesh = plsc.VectorSubcoreMesh(
      core_axis_name="c", subcore_axis_name="s", num_cores=NC, num_subcores=NS)

  def body(tbl, srcf, dstf, zeros, out, sbuf, dbuf, rows, acc, gsem, isem,
           ssem):
    c = lax.axis_index("c")
    s = lax.axis_index("s")
    tail0 = (NS - 1) * RS
    tail_n = N - tail0

    def copy_rows(mk_src, mk_dst):
      r0 = s * RS

      @pl.when(s < NS - 1)
      def _():
        pltpu.sync_copy(mk_src(r0, RS), mk_dst(r0, RS))

      @pl.when(s == NS - 1)
      def _():
        pltpu.sync_copy(mk_src(tail0, tail_n), mk_dst(tail0, tail_n))

    copy_rows(lambda r, n: zeros.at[pl.ds(r, n)],
              lambda r, n: acc.at[pl.ds(r, n)])
    plsc.subcore_barrier()

    base = (c * NS + s) * npc * CH

    def load_idx(j, jm4):
      off = base + j * CH
      pltpu.async_copy(srcf.at[pl.ds(off, CH)], sbuf.at[jm4], isem)
      pltpu.async_copy(dstf.at[pl.ds(off, CH)], dbuf.at[jm4], isem)

    def wait_idx_pair():
      pltpu.make_async_copy(srcf.at[pl.ds(0, CH)], sbuf.at[0], isem).wait()
      pltpu.make_async_copy(dstf.at[pl.ds(0, CH)], dbuf.at[0], isem).wait()

    def gather(jm4, jm3):
      pltpu.async_copy(tbl.at[sbuf.at[jm4]], rows.at[jm3], gsem)

    def wait_gather():
      pltpu.make_async_copy(tbl.at[sbuf.at[0]], rows.at[0], gsem).wait()

    def scatter(jm4, jm3):
      pltpu.async_copy(rows.at[jm3], acc.at[dbuf.at[jm4]], ssem, add=True)

    def wait_scatter():
      pltpu.make_async_copy(rows.at[0], acc.at[dbuf.at[0]], ssem).wait()

    load_idx(0, 0)
    wait_idx_pair()
    gather(0, 0)
    load_idx(1, 1)
    wait_idx_pair()
    gather(1, 1)
    load_idx(2, 2)
    wait_idx_pair()
    gather(2, 2)
    load_idx(3, 3)

    def it(j, carry):
      wait_gather()

      @pl.when(j >= 1)
      def _():
        wait_scatter()

      @pl.when(j + 3 < npc)
      def _():
        wait_idx_pair()
        gather((j + 3) % 5, (j + 3) % 4)

      @pl.when(j + 4 < npc)
      def _():
        load_idx(j + 4, (j + 4) % 5)

      scatter(j % 5, j % 4)
      return carry

    lax.fori_loop(0, npc, it, 0)
    wait_scatter()
    plsc.subcore_barrier()
    copy_rows(lambda r, n: acc.at[pl.ds(r, n)],
              lambda r, n: out.at[c, pl.ds(r, n)])

  return pl.kernel(
      body,
      out_type=jax.ShapeDtypeStruct((NC, N, 128), jnp.float32),
      mesh=mesh,
      scratch_types=[
          pltpu.VMEM((5, CH), jnp.int32),
          pltpu.VMEM((5, CH), jnp.int32),
          pltpu.VMEM((4, CH, 128), jnp.float32),
          pltpu.VMEM_SHARED((N + NTRASH, 128), jnp.float32),
          pltpu.SemaphoreType.DMA,
          pltpu.SemaphoreType.DMA,
          pltpu.SemaphoreType.DMA,
      ],
  )


NPC_EDGE = 114
NPC_FEAT = 228
_sc_agg_edge = _make_sc_agg(NPC_EDGE)
_sc_agg_feat = _make_sc_agg(NPC_FEAT)


def _pad_idx(parts_src, parts_dst, per_core):
  pad = per_core - parts_src[0][0].shape[0]
  ar = jnp.arange(pad, dtype=jnp.int32)
  psrc = (ar * 131) % N
  pdst = N + (ar % NTRASH)
  srcf = jnp.concatenate(
      [jnp.concatenate([p, psrc + off]) for p, off in parts_src])
  dstf = jnp.concatenate([jnp.concatenate([p, pdst]) for p in parts_dst])
  return (srcf, dstf)


def _mlp(z, w1, b1, w2, b2):
  a = jnp.maximum(jnp.dot(z, w1, preferred_element_type=jnp.float32) + b1, 0.0)
  return jnp.maximum(jnp.dot(a, w2, preferred_element_type=jnp.float32) + b2,
                     0.0)


def _l0_body(eps_r, x_r, agg_r, w1_r, b1_r, w2_r, b2_r, out_r):
  z = x_r[...] * (1.0 + eps_r[0, 0]) + agg_r[0] + agg_r[1]
  o = _mlp(z, w1_r[...], b1_r[...], w2_r[...], b2_r[...])
  out_r[0] = o[:, :128]
  out_r[1] = o[:, 128:]


def _mid_body(eps_r, h_r, agg_r, w1_r, b1_r, w2_r, b2_r, out_r):
  h = jnp.concatenate([h_r[0], h_r[1]], axis=1)
  ag = jnp.concatenate([agg_r[0], agg_r[1]], axis=1)
  o = _mlp(h * (1.0 + eps_r[0, 0]) + ag, w1_r[...], b1_r[...], w2_r[...],
           b2_r[...])
  out_r[0] = o[:, :128]
  out_r[1] = o[:, 128:]


def _last_body(eps_r, h_r, agg_r, b_r, w1_r, b1_r, w2_r, b2_r,
               pooled_r, cnt_r):
  i = pl.program_id(0)
  h = jnp.concatenate([h_r[0], h_r[1]], axis=1)
  ag = jnp.concatenate([agg_r[0], agg_r[1]], axis=1)
  o = _mlp(h * (1.0 + eps_r[0, 0]) + ag, w1_r[...], b1_r[...], w2_r[...],
           b2_r[...])
  gids = lax.broadcasted_iota(jnp.int32, (N_GRAPH, 1), 0)
  oh = (gids == b_r[0]).astype(jnp.float32)
  ps = jnp.dot(oh, o, preferred_element_type=jnp.float32)
  cs = jnp.broadcast_to(jnp.sum(oh, axis=1, keepdims=True), (N_GRAPH, 128))

  @pl.when(i == 0)
  def _():
    pooled_r[...] = ps
    cnt_r[...] = cs

  @pl.when(i > 0)
  def _():
    pooled_r[...] += ps
    cnt_r[...] += cs


_smem11 = pl.BlockSpec(memory_space=pltpu.SMEM)
_half_spec = pl.BlockSpec((2, BLK, 128), lambda i: (0, i, 0))


def _tc_l0(eps_e, x, agg, w1, b1, w2, b2):
  return pl.pallas_call(
      _l0_body,
      grid=(NB,),
      in_specs=[
          _smem11,
          pl.BlockSpec((BLK, 128), lambda i: (i, 0)),
          _half_spec,
          pl.BlockSpec((128, HID), lambda i: (0, 0)),
          pl.BlockSpec((1, HID), lambda i: (0, 0)),
          pl.BlockSpec((HID, HID), lambda i: (0, 0)),
          pl.BlockSpec((1, HID), lambda i: (0, 0)),
      ],
      out_specs=_half_spec,
      out_shape=jax.ShapeDtypeStruct((2, N, 128), jnp.float32),
  )(eps_e, x, agg, w1, b1, w2, b2)


def _tc_mid(eps_e, h, agg, w1, b1, w2, b2):
  return pl.pallas_call(
      _mid_body,
      grid=(NB,),
      in_specs=[
          _smem11,
          _half_spec,
          _half_spec,
          pl.BlockSpec((HID, HID), lambda i: (0, 0)),
          pl.BlockSpec((1, HID), lambda i: (0, 0)),
          pl.BlockSpec((HID, HID), lambda i: (0, 0)),
          pl.BlockSpec((1, HID), lambda i: (0, 0)),
      ],
      out_specs=_half_spec,
      out_shape=jax.ShapeDtypeStruct((2, N, 128), jnp.float32),
  )(eps_e, h, agg, w1, b1, w2, b2)


def _tc_last(eps_e, h, agg, batch3, w1, b1, w2, b2):
  return pl.pallas_call(
      _last_body,
      grid=(NB,),
      in_specs=[
          _smem11,
          _half_spec,
          _half_spec,
          pl.BlockSpec((1, 1, BLK), lambda i: (i, 0, 0)),
          pl.BlockSpec((HID, HID), lambda i: (0, 0)),
          pl.BlockSpec((1, HID), lambda i: (0, 0)),
          pl.BlockSpec((HID, HID), lambda i: (0, 0)),
          pl.BlockSpec((1, HID), lambda i: (0, 0)),
      ],
      out_specs=[
          pl.BlockSpec((N_GRAPH, HID), lambda i: (0, 0)),
          pl.BlockSpec((N_GRAPH, 128), lambda i: (0, 0)),
      ],
      out_shape=[
          jax.ShapeDtypeStruct((N_GRAPH, HID), jnp.float32),
          jax.ShapeDtypeStruct((N_GRAPH, 128), jnp.float32),
      ],
  )(eps_e, h, agg, batch3, w1, b1, w2, b2)


def _head_body(cnt_r, wo_r, bo_r, p0, p1, p2, p3, out_r):
  inv = 1.0 / jnp.maximum(cnt_r[...][:, 0:1], 1.0)
  bo = bo_r[...]
  acc = jnp.zeros((N_GRAPH, N_CLS), jnp.float32)
  for e, p in enumerate((p0, p1, p2, p3)):
    acc = acc + jnp.dot(p[...] * inv, wo_r[e],
                        preferred_element_type=jnp.float32) + bo[e:e + 1, :]
  out_r[...] = acc * 0.25


def _head(cnt, wout, bout, pooled):
  return pl.pallas_call(
      _head_body,
      out_shape=jax.ShapeDtypeStruct((N_GRAPH, N_CLS), jnp.float32),
  )(cnt, wout, bout, *pooled)


def kernel(x, edge_index, batch, W1_0, b1_0, W2_0, b2_0, W1_1, b1_1, W2_1,
           b2_1, W1_2, b1_2, W2_2, b2_2, eps, Wout, bout):
  src = edge_index[0]
  dst = edge_index[1]
  half = E // NC
  src_e, dst_e = _pad_idx(
      [(src[:half], 0), (src[half:], 0)],
      [dst[:half], dst[half:]], NS * NPC_EDGE * CH)
  src_f, dst_f = _pad_idx(
      [(src, 0), (src + N, N)],
      [dst, dst], NS * NPC_FEAT * CH)
  zeros = jnp.zeros((N, 128), jnp.float32)
  batch3 = batch.reshape(NB, 1, BLK)

  agg0 = _sc_agg_edge(x, src_e, dst_e, zeros)

  h1, h2, agg1, agg2, pooled = [], [], [], [], []
  cnt = None
  for e in range(N_EXP):
    h1.append(_tc_l0(eps[e, 0].reshape(1, 1), x, agg0, W1_0[e],
                     b1_0[e].reshape(1, HID), W2_0[e], b2_0[e].reshape(1, HID)))
  for e in range(N_EXP):
    agg1.append(_sc_agg_feat(h1[e].reshape(2 * N, 128), src_f, dst_f, zeros))
  for e in range(N_EXP):
    h2.append(_tc_mid(eps[e, 1].reshape(1, 1), h1[e], agg1[e], W1_1[e],
                      b1_1[e].reshape(1, HID), W2_1[e],
                      b2_1[e].reshape(1, HID)))
  for e in range(N_EXP):
    agg2.append(_sc_agg_feat(h2[e].reshape(2 * N, 128), src_f, dst_f, zeros))
  for e in range(N_EXP):
    p, c = _tc_last(eps[e, 2].reshape(1, 1), h2[e], agg2[e], batch3, W1_2[e],
                    b1_2[e].reshape(1, HID), W2_2[e], b2_2[e].reshape(1, HID))
    pooled.append(p)
    if cnt is None:
      cnt = c
  return _head(cnt, Wout, bout, pooled)


# speedup vs baseline: 1.4660x; 1.0028x over previous
import functools

import jax
import jax.numpy as jnp
from jax import lax
from jax.experimental import pallas as pl
from jax.experimental.pallas import tpu as pltpu
from jax.experimental.pallas import tpu_sc as plsc

N = 10000
E = 320000
F_IN = 128
HID = 256
N_EXP = 4
N_GRAPH = 64
N_CLS = 10

NC = 2
NS = 16
CH = 88
NTRASH = 64
RS = 632

NB = 10
BLK = N // NB


def _make_sc_agg(npc):
  mesh = plsc.VectorSubcoreMesh(
      core_axis_name="c", subcore_axis_name="s", num_cores=NC, num_subcores=NS)

  def body(tbl, srcf, dstf, zeros, out, sbuf, dbuf, rows, acc, gsem, isem,
           ssem):
    c = lax.axis_index("c")
    s = lax.axis_index("s")
    tail0 = (NS - 1) * RS
    tail_n = N - tail0

    def copy_rows(mk_src, mk_dst):
      r0 = s * RS

      @pl.when(s < NS - 1)
      def _():
        pltpu.sync_copy(mk_src(r0, RS), mk_dst(r0, RS))

      @pl.when(s == NS - 1)
      def _():
        pltpu.sync_copy(mk_src(tail0, tail_n), mk_dst(tail0, tail_n))

    base = (c * NS + s) * npc * CH

    def load_idx(j, jm4):
      off = base + j * CH
      pltpu.async_copy(srcf.at[pl.ds(off, CH)], sbuf.at[jm4], isem)
      pltpu.async_copy(dstf.at[pl.ds(off, CH)], dbuf.at[jm4], isem)

    def wait_idx_pair():
      pltpu.make_async_copy(srcf.at[pl.ds(0, CH)], sbuf.at[0], isem).wait()
      pltpu.make_async_copy(dstf.at[pl.ds(0, CH)], dbuf.at[0], isem).wait()

    def gather(jm4, jm3):
      pltpu.async_copy(tbl.at[sbuf.at[jm4]], rows.at[jm3], gsem)

    def wait_gather():
      pltpu.make_async_copy(tbl.at[sbuf.at[0]], rows.at[0], gsem).wait()

    def scatter(jm4, jm3):
      pltpu.async_copy(rows.at[jm3], acc.at[dbuf.at[jm4]], ssem, add=True)

    def wait_scatter():
      pltpu.make_async_copy(rows.at[0], acc.at[dbuf.at[0]], ssem).wait()

    load_idx(0, 0)
    wait_idx_pair()
    gather(0, 0)
    load_idx(1, 1)
    wait_idx_pair()
    gather(1, 1)
    load_idx(2, 2)
    wait_idx_pair()
    gather(2, 2)
    load_idx(3, 3)

    copy_rows(lambda r, n: zeros.at[pl.ds(r, n)],
              lambda r, n: acc.at[pl.ds(r, n)])
    plsc.subcore_barrier()

    def it(j, carry):
      wait_gather()

      @pl.when(j >= 1)
      def _():
        wait_scatter()

      @pl.when(j + 3 < npc)
      def _():
        wait_idx_pair()
        gather((j + 3) % 5, (j + 3) % 4)

      @pl.when(j + 4 < npc)
      def _():
        load_idx(j + 4, (j + 4) % 5)

      scatter(j % 5, j % 4)
      return carry

    lax.fori_loop(0, npc, it, 0)
    wait_scatter()
    plsc.subcore_barrier()
    copy_rows(lambda r, n: acc.at[pl.ds(r, n)],
              lambda r, n: out.at[c, pl.ds(r, n)])

  return pl.kernel(
      body,
      out_type=jax.ShapeDtypeStruct((NC, N, 128), jnp.float32),
      mesh=mesh,
      scratch_types=[
          pltpu.VMEM((5, CH), jnp.int32),
          pltpu.VMEM((5, CH), jnp.int32),
          pltpu.VMEM((4, CH, 128), jnp.float32),
          pltpu.VMEM_SHARED((N + NTRASH, 128), jnp.float32),
          pltpu.SemaphoreType.DMA,
          pltpu.SemaphoreType.DMA,
          pltpu.SemaphoreType.DMA,
      ],
  )


NPC_EDGE = 114
NPC_FEAT = 228
_sc_agg_edge = _make_sc_agg(NPC_EDGE)
_sc_agg_feat = _make_sc_agg(NPC_FEAT)


def _pad_idx(parts_src, parts_dst, per_core):
  pad = per_core - parts_src[0][0].shape[0]
  ar = jnp.arange(pad, dtype=jnp.int32)
  psrc = (ar * 131) % N
  pdst = N + (ar % NTRASH)
  srcf = jnp.concatenate(
      [jnp.concatenate([p, psrc + off]) for p, off in parts_src])
  dstf = jnp.concatenate([jnp.concatenate([p, pdst]) for p in parts_dst])
  return (srcf, dstf)


def _mlp(z, w1, b1, w2, b2):
  a = jnp.maximum(jnp.dot(z, w1, preferred_element_type=jnp.float32) + b1, 0.0)
  return jnp.maximum(jnp.dot(a, w2, preferred_element_type=jnp.float32) + b2,
                     0.0)


def _l0_body(eps_r, x_r, agg_r, w1_r, b1_r, w2_r, b2_r, out_r):
  z = x_r[...] * (1.0 + eps_r[0, 0]) + agg_r[0] + agg_r[1]
  o = _mlp(z, w1_r[...], b1_r[...], w2_r[...], b2_r[...])
  out_r[0] = o[:, :128]
  out_r[1] = o[:, 128:]


def _mid_body(eps_r, h_r, agg_r, w1_r, b1_r, w2_r, b2_r, out_r):
  h = jnp.concatenate([h_r[0], h_r[1]], axis=1)
  ag = jnp.concatenate([agg_r[0], agg_r[1]], axis=1)
  o = _mlp(h * (1.0 + eps_r[0, 0]) + ag, w1_r[...], b1_r[...], w2_r[...],
           b2_r[...])
  out_r[0] = o[:, :128]
  out_r[1] = o[:, 128:]


def _last_body(eps_r, h_r, agg_r, b_r, w1_r, b1_r, w2_r, b2_r,
               pooled_r, cnt_r):
  i = pl.program_id(0)
  h = jnp.concatenate([h_r[0], h_r[1]], axis=1)
  ag = jnp.concatenate([agg_r[0], agg_r[1]], axis=1)
  o = _mlp(h * (1.0 + eps_r[0, 0]) + ag, w1_r[...], b1_r[...], w2_r[...],
           b2_r[...])
  gids = lax.broadcasted_iota(jnp.int32, (N_GRAPH, 1), 0)
  oh = (gids == b_r[0]).astype(jnp.float32)
  ps = jnp.dot(oh, o, preferred_element_type=jnp.float32)
  cs = jnp.broadcast_to(jnp.sum(oh, axis=1, keepdims=True), (N_GRAPH, 128))

  @pl.when(i == 0)
  def _():
    pooled_r[...] = ps
    cnt_r[...] = cs

  @pl.when(i > 0)
  def _():
    pooled_r[...] += ps
    cnt_r[...] += cs


_smem11 = pl.BlockSpec(memory_space=pltpu.SMEM)
_half_spec = pl.BlockSpec((2, BLK, 128), lambda i: (0, i, 0))


def _tc_l0(eps_e, x, agg, w1, b1, w2, b2):
  return pl.pallas_call(
      _l0_body,
      grid=(NB,),
      in_specs=[
          _smem11,
          pl.BlockSpec((BLK, 128), lambda i: (i, 0)),
          _half_spec,
          pl.BlockSpec((128, HID), lambda i: (0, 0)),
          pl.BlockSpec((1, HID), lambda i: (0, 0)),
          pl.BlockSpec((HID, HID), lambda i: (0, 0)),
          pl.BlockSpec((1, HID), lambda i: (0, 0)),
      ],
      out_specs=_half_spec,
      out_shape=jax.ShapeDtypeStruct((2, N, 128), jnp.float32),
  )(eps_e, x, agg, w1, b1, w2, b2)


def _tc_mid(eps_e, h, agg, w1, b1, w2, b2):
  return pl.pallas_call(
      _mid_body,
      grid=(NB,),
      in_specs=[
          _smem11,
          _half_spec,
          _half_spec,
          pl.BlockSpec((HID, HID), lambda i: (0, 0)),
          pl.BlockSpec((1, HID), lambda i: (0, 0)),
          pl.BlockSpec((HID, HID), lambda i: (0, 0)),
          pl.BlockSpec((1, HID), lambda i: (0, 0)),
      ],
      out_specs=_half_spec,
      out_shape=jax.ShapeDtypeStruct((2, N, 128), jnp.float32),
  )(eps_e, h, agg, w1, b1, w2, b2)


def _tc_last(eps_e, h, agg, batch3, w1, b1, w2, b2):
  return pl.pallas_call(
      _last_body,
      grid=(NB,),
      in_specs=[
          _smem11,
          _half_spec,
          _half_spec,
          pl.BlockSpec((1, 1, BLK), lambda i: (i, 0, 0)),
          pl.BlockSpec((HID, HID), lambda i: (0, 0)),
          pl.BlockSpec((1, HID), lambda i: (0, 0)),
          pl.BlockSpec((HID, HID), lambda i: (0, 0)),
          pl.BlockSpec((1, HID), lambda i: (0, 0)),
      ],
      out_specs=[
          pl.BlockSpec((N_GRAPH, HID), lambda i: (0, 0)),
          pl.BlockSpec((N_GRAPH, 128), lambda i: (0, 0)),
      ],
      out_shape=[
          jax.ShapeDtypeStruct((N_GRAPH, HID), jnp.float32),
          jax.ShapeDtypeStruct((N_GRAPH, 128), jnp.float32),
      ],
  )(eps_e, h, agg, batch3, w1, b1, w2, b2)


def _head_body(cnt_r, wo_r, bo_r, p0, p1, p2, p3, out_r):
  inv = 1.0 / jnp.maximum(cnt_r[...][:, 0:1], 1.0)
  bo = bo_r[...]
  acc = jnp.zeros((N_GRAPH, N_CLS), jnp.float32)
  for e, p in enumerate((p0, p1, p2, p3)):
    acc = acc + jnp.dot(p[...] * inv, wo_r[e],
                        preferred_element_type=jnp.float32) + bo[e:e + 1, :]
  out_r[...] = acc * 0.25


def _head(cnt, wout, bout, pooled):
  return pl.pallas_call(
      _head_body,
      out_shape=jax.ShapeDtypeStruct((N_GRAPH, N_CLS), jnp.float32),
  )(cnt, wout, bout, *pooled)


def kernel(x, edge_index, batch, W1_0, b1_0, W2_0, b2_0, W1_1, b1_1, W2_1,
           b2_1, W1_2, b1_2, W2_2, b2_2, eps, Wout, bout):
  src = edge_index[0]
  dst = edge_index[1]
  half = E // NC
  src_e, dst_e = _pad_idx(
      [(src[:half], 0), (src[half:], 0)],
      [dst[:half], dst[half:]], NS * NPC_EDGE * CH)
  src_f, dst_f = _pad_idx(
      [(src, 0), (src + N, N)],
      [dst, dst], NS * NPC_FEAT * CH)
  zeros = jnp.zeros((N, 128), jnp.float32)
  batch3 = batch.reshape(NB, 1, BLK)

  agg0 = _sc_agg_edge(x, src_e, dst_e, zeros)

  h1, h2, agg1, agg2, pooled = [], [], [], [], []
  cnt = None
  for e in range(N_EXP):
    h1.append(_tc_l0(eps[e, 0].reshape(1, 1), x, agg0, W1_0[e],
                     b1_0[e].reshape(1, HID), W2_0[e], b2_0[e].reshape(1, HID)))
  for e in range(N_EXP):
    agg1.append(_sc_agg_feat(h1[e].reshape(2 * N, 128), src_f, dst_f, zeros))
  for e in range(N_EXP):
    h2.append(_tc_mid(eps[e, 1].reshape(1, 1), h1[e], agg1[e], W1_1[e],
                      b1_1[e].reshape(1, HID), W2_1[e],
                      b2_1[e].reshape(1, HID)))
  for e in range(N_EXP):
    agg2.append(_sc_agg_feat(h2[e].reshape(2 * N, 128), src_f, dst_f, zeros))
  for e in range(N_EXP):
    p, c = _tc_last(eps[e, 2].reshape(1, 1), h2[e], agg2[e], batch3, W1_2[e],
                    b1_2[e].reshape(1, HID), W2_2[e], b2_2[e].reshape(1, HID))
    pooled.append(p)
    if cnt is None:
      cnt = c
  return _head(cnt, Wout, bout, pooled)


# speedup vs baseline: 1.4668x; 1.0005x over previous
import functools

import jax
import jax.numpy as jnp
from jax import lax
from jax.experimental import pallas as pl
from jax.experimental.pallas import tpu as pltpu
from jax.experimental.pallas import tpu_sc as plsc

N = 10000
E = 320000
F_IN = 128
HID = 256
N_EXP = 4
N_GRAPH = 64
N_CLS = 10

NC = 2
NS = 16
CH = 88
NTRASH = 64
RS = 632

NB = 10
BLK = N // NB


def _make_sc_agg(npc):
  mesh = plsc.VectorSubcoreMesh(
      core_axis_name="c", subcore_axis_name="s", num_cores=NC, num_subcores=NS)

  def body(tbl, srcf, dstf, zeros, out, sbuf, dbuf, rows, acc, gsem, isem,
           ssem):
    c = lax.axis_index("c")
    s = lax.axis_index("s")
    tail0 = (NS - 1) * RS
    tail_n = N - tail0

    def copy_rows(mk_src, mk_dst):
      r0 = s * RS

      @pl.when(s < NS - 1)
      def _():
        pltpu.sync_copy(mk_src(r0, RS), mk_dst(r0, RS))

      @pl.when(s == NS - 1)
      def _():
        pltpu.sync_copy(mk_src(tail0, tail_n), mk_dst(tail0, tail_n))

    base = (c * NS + s) * npc * CH

    def load_idx(j, jm4):
      off = base + j * CH
      pltpu.async_copy(srcf.at[pl.ds(off, CH)], sbuf.at[jm4], isem)
      pltpu.async_copy(dstf.at[pl.ds(off, CH)], dbuf.at[jm4], isem)

    def wait_idx_pair():
      pltpu.make_async_copy(srcf.at[pl.ds(0, CH)], sbuf.at[0], isem).wait()
      pltpu.make_async_copy(dstf.at[pl.ds(0, CH)], dbuf.at[0], isem).wait()

    def gather(jm4, jm3):
      pltpu.async_copy(tbl.at[sbuf.at[jm4]], rows.at[jm3], gsem)

    def wait_gather():
      pltpu.make_async_copy(tbl.at[sbuf.at[0]], rows.at[0], gsem).wait()

    def scatter(jm4, jm3):
      pltpu.async_copy(rows.at[jm3], acc.at[dbuf.at[jm4]], ssem, add=True)

    def wait_scatter():
      pltpu.make_async_copy(rows.at[0], acc.at[dbuf.at[0]], ssem).wait()

    load_idx(0, 0)
    wait_idx_pair()
    gather(0, 0)
    load_idx(1, 1)
    wait_idx_pair()
    gather(1, 1)
    load_idx(2, 2)
    wait_idx_pair()
    gather(2, 2)
    load_idx(3, 3)

    copy_rows(lambda r, n: zeros.at[pl.ds(r, n)],
              lambda r, n: acc.at[pl.ds(r, n)])
    plsc.subcore_barrier()

    def it(j, carry):
      wait_gather()

      @pl.when(j >= 1)
      def _():
        wait_scatter()

      scatter(j % 5, j % 4)

      @pl.when(j + 3 < npc)
      def _():
        wait_idx_pair()
        gather((j + 3) % 5, (j + 3) % 4)

      @pl.when(j + 4 < npc)
      def _():
        load_idx(j + 4, (j + 4) % 5)

      return carry

    lax.fori_loop(0, npc, it, 0)
    wait_scatter()
    plsc.subcore_barrier()
    copy_rows(lambda r, n: acc.at[pl.ds(r, n)],
              lambda r, n: out.at[c, pl.ds(r, n)])

  return pl.kernel(
      body,
      out_type=jax.ShapeDtypeStruct((NC, N, 128), jnp.float32),
      mesh=mesh,
      scratch_types=[
          pltpu.VMEM((5, CH), jnp.int32),
          pltpu.VMEM((5, CH), jnp.int32),
          pltpu.VMEM((4, CH, 128), jnp.float32),
          pltpu.VMEM_SHARED((N + NTRASH, 128), jnp.float32),
          pltpu.SemaphoreType.DMA,
          pltpu.SemaphoreType.DMA,
          pltpu.SemaphoreType.DMA,
      ],
  )


NPC_EDGE = 114
NPC_FEAT = 228
_sc_agg_edge = _make_sc_agg(NPC_EDGE)
_sc_agg_feat = _make_sc_agg(NPC_FEAT)


def _pad_idx(parts_src, parts_dst, per_core):
  pad = per_core - parts_src[0][0].shape[0]
  ar = jnp.arange(pad, dtype=jnp.int32)
  psrc = (ar * 131) % N
  pdst = N + (ar % NTRASH)
  srcf = jnp.concatenate(
      [jnp.concatenate([p, psrc + off]) for p, off in parts_src])
  dstf = jnp.concatenate([jnp.concatenate([p, pdst]) for p in parts_dst])
  return (srcf, dstf)


def _mlp(z, w1, b1, w2, b2):
  a = jnp.maximum(jnp.dot(z, w1, preferred_element_type=jnp.float32) + b1, 0.0)
  return jnp.maximum(jnp.dot(a, w2, preferred_element_type=jnp.float32) + b2,
                     0.0)


def _l0_body(eps_r, x_r, agg_r, w1_r, b1_r, w2_r, b2_r, out_r):
  z = x_r[...] * (1.0 + eps_r[0, 0]) + agg_r[0] + agg_r[1]
  o = _mlp(z, w1_r[...], b1_r[...], w2_r[...], b2_r[...])
  out_r[0] = o[:, :128]
  out_r[1] = o[:, 128:]


def _mid_body(eps_r, h_r, agg_r, w1_r, b1_r, w2_r, b2_r, out_r):
  h = jnp.concatenate([h_r[0], h_r[1]], axis=1)
  ag = jnp.concatenate([agg_r[0], agg_r[1]], axis=1)
  o = _mlp(h * (1.0 + eps_r[0, 0]) + ag, w1_r[...], b1_r[...], w2_r[...],
           b2_r[...])
  out_r[0] = o[:, :128]
  out_r[1] = o[:, 128:]


def _last_body(eps_r, h_r, agg_r, b_r, w1_r, b1_r, w2_r, b2_r,
               pooled_r, cnt_r):
  i = pl.program_id(0)
  h = jnp.concatenate([h_r[0], h_r[1]], axis=1)
  ag = jnp.concatenate([agg_r[0], agg_r[1]], axis=1)
  o = _mlp(h * (1.0 + eps_r[0, 0]) + ag, w1_r[...], b1_r[...], w2_r[...],
           b2_r[...])
  gids = lax.broadcasted_iota(jnp.int32, (N_GRAPH, 1), 0)
  oh = (gids == b_r[0]).astype(jnp.float32)
  ps = jnp.dot(oh, o, preferred_element_type=jnp.float32)
  cs = jnp.broadcast_to(jnp.sum(oh, axis=1, keepdims=True), (N_GRAPH, 128))

  @pl.when(i == 0)
  def _():
    pooled_r[...] = ps
    cnt_r[...] = cs

  @pl.when(i > 0)
  def _():
    pooled_r[...] += ps
    cnt_r[...] += cs


_smem11 = pl.BlockSpec(memory_space=pltpu.SMEM)
_half_spec = pl.BlockSpec((2, BLK, 128), lambda i: (0, i, 0))


def _tc_l0(eps_e, x, agg, w1, b1, w2, b2):
  return pl.pallas_call(
      _l0_body,
      grid=(NB,),
      in_specs=[
          _smem11,
          pl.BlockSpec((BLK, 128), lambda i: (i, 0)),
          _half_spec,
          pl.BlockSpec((128, HID), lambda i: (0, 0)),
          pl.BlockSpec((1, HID), lambda i: (0, 0)),
          pl.BlockSpec((HID, HID), lambda i: (0, 0)),
          pl.BlockSpec((1, HID), lambda i: (0, 0)),
      ],
      out_specs=_half_spec,
      out_shape=jax.ShapeDtypeStruct((2, N, 128), jnp.float32),
  )(eps_e, x, agg, w1, b1, w2, b2)


def _tc_mid(eps_e, h, agg, w1, b1, w2, b2):
  return pl.pallas_call(
      _mid_body,
      grid=(NB,),
      in_specs=[
          _smem11,
          _half_spec,
          _half_spec,
          pl.BlockSpec((HID, HID), lambda i: (0, 0)),
          pl.BlockSpec((1, HID), lambda i: (0, 0)),
          pl.BlockSpec((HID, HID), lambda i: (0, 0)),
          pl.BlockSpec((1, HID), lambda i: (0, 0)),
      ],
      out_specs=_half_spec,
      out_shape=jax.ShapeDtypeStruct((2, N, 128), jnp.float32),
  )(eps_e, h, agg, w1, b1, w2, b2)


def _tc_last(eps_e, h, agg, batch3, w1, b1, w2, b2):
  return pl.pallas_call(
      _last_body,
      grid=(NB,),
      in_specs=[
          _smem11,
          _half_spec,
          _half_spec,
          pl.BlockSpec((1, 1, BLK), lambda i: (i, 0, 0)),
          pl.BlockSpec((HID, HID), lambda i: (0, 0)),
          pl.BlockSpec((1, HID), lambda i: (0, 0)),
          pl.BlockSpec((HID, HID), lambda i: (0, 0)),
          pl.BlockSpec((1, HID), lambda i: (0, 0)),
      ],
      out_specs=[
          pl.BlockSpec((N_GRAPH, HID), lambda i: (0, 0)),
          pl.BlockSpec((N_GRAPH, 128), lambda i: (0, 0)),
      ],
      out_shape=[
          jax.ShapeDtypeStruct((N_GRAPH, HID), jnp.float32),
          jax.ShapeDtypeStruct((N_GRAPH, 128), jnp.float32),
      ],
  )(eps_e, h, agg, batch3, w1, b1, w2, b2)


def _head_body(cnt_r, wo_r, bo_r, p0, p1, p2, p3, out_r):
  inv = 1.0 / jnp.maximum(cnt_r[...][:, 0:1], 1.0)
  bo = bo_r[...]
  acc = jnp.zeros((N_GRAPH, N_CLS), jnp.float32)
  for e, p in enumerate((p0, p1, p2, p3)):
    acc = acc + jnp.dot(p[...] * inv, wo_r[e],
                        preferred_element_type=jnp.float32) + bo[e:e + 1, :]
  out_r[...] = acc * 0.25


def _head(cnt, wout, bout, pooled):
  return pl.pallas_call(
      _head_body,
      out_shape=jax.ShapeDtypeStruct((N_GRAPH, N_CLS), jnp.float32),
  )(cnt, wout, bout, *pooled)


def kernel(x, edge_index, batch, W1_0, b1_0, W2_0, b2_0, W1_1, b1_1, W2_1,
           b2_1, W1_2, b1_2, W2_2, b2_2, eps, Wout, bout):
  src = edge_index[0]
  dst = edge_index[1]
  half = E // NC
  src_e, dst_e = _pad_idx(
      [(src[:half], 0), (src[half:], 0)],
      [dst[:half], dst[half:]], NS * NPC_EDGE * CH)
  src_f, dst_f = _pad_idx(
      [(src, 0), (src + N, N)],
      [dst, dst], NS * NPC_FEAT * CH)
  zeros = jnp.zeros((N, 128), jnp.float32)
  batch3 = batch.reshape(NB, 1, BLK)

  agg0 = _sc_agg_edge(x, src_e, dst_e, zeros)

  h1, h2, agg1, agg2, pooled = [], [], [], [], []
  cnt = None
  for e in range(N_EXP):
    h1.append(_tc_l0(eps[e, 0].reshape(1, 1), x, agg0, W1_0[e],
                     b1_0[e].reshape(1, HID), W2_0[e], b2_0[e].reshape(1, HID)))
  for e in range(N_EXP):
    agg1.append(_sc_agg_feat(h1[e].reshape(2 * N, 128), src_f, dst_f, zeros))
  for e in range(N_EXP):
    h2.append(_tc_mid(eps[e, 1].reshape(1, 1), h1[e], agg1[e], W1_1[e],
                      b1_1[e].reshape(1, HID), W2_1[e],
                      b2_1[e].reshape(1, HID)))
  for e in range(N_EXP):
    agg2.append(_sc_agg_feat(h2[e].reshape(2 * N, 128), src_f, dst_f, zeros))
  for e in range(N_EXP):
    p, c = _tc_last(eps[e, 2].reshape(1, 1), h2[e], agg2[e], batch3, W1_2[e],
                    b1_2[e].reshape(1, HID), W2_2[e], b2_2[e].reshape(1, HID))
    pooled.append(p)
    if cnt is None:
      cnt = c
  return _head(cnt, Wout, bout, pooled)
